# baseline (speedup 1.0000x reference)
import functools

import jax
import jax.numpy as jnp
from jax import lax
from jax.experimental import pallas as pl
from jax.experimental.pallas import tpu as pltpu
from jax.experimental.pallas import tpu_sc as plsc

N = 10000
NP = 10240
E = 160000
EPAD = 163840
D = 256
H = 256
L = 3
EPS = 1e-5

NS = 16
ROWS_PER_TILE = NP // NS
CH = 80
AGG_CHUNKS = EPAD // (NS * CH)
DEG_CHUNKS = AGG_CHUNKS // 2

_MESH = plsc.VectorSubcoreMesh(core_axis_name="c", subcore_axis_name="s")


@functools.partial(
    pl.kernel,
    mesh=_MESH,
    out_type=jax.ShapeDtypeStruct((2, NP), jnp.float32),
    scratch_types=[
        pltpu.VMEM((DEG_CHUNKS, CH), jnp.int32),
        pltpu.VMEM((CH,), jnp.float32),
        pltpu.VMEM((ROWS_PER_TILE,), jnp.float32),
        pltpu.VMEM_SHARED((NP,), jnp.float32),
    ],
)
def _sc_degree(col_hbm, deg_hbm, colv, onesv, wbuf, shared):
    c = lax.axis_index("c")
    s = lax.axis_index("s")
    one = jnp.ones((16,), jnp.float32)
    zero = jnp.zeros((16,), jnp.float32)
    for j in range(CH // 16):
        onesv[pl.ds(16 * j, 16)] = one
    for j in range(ROWS_PER_TILE // 16):
        wbuf[pl.ds(16 * j, 16)] = zero
    base = s * ROWS_PER_TILE
    pltpu.sync_copy(wbuf, shared.at[pl.ds(base, ROWS_PER_TILE)])
    pltpu.sync_copy(col_hbm.at[s, pl.ds(c * DEG_CHUNKS, DEG_CHUNKS)], colv)
    plsc.subcore_barrier()

    def body(j, carry):
        pltpu.sync_copy(onesv, shared.at[colv.at[j]], add=True)
        return carry

    lax.fori_loop(0, DEG_CHUNKS, body, 0)
    plsc.subcore_barrier()
    pltpu.sync_copy(shared.at[pl.ds(base, ROWS_PER_TILE)], wbuf)
    pltpu.sync_copy(wbuf, deg_hbm.at[c, pl.ds(base, ROWS_PER_TILE)])


@functools.partial(
    pl.kernel,
    mesh=_MESH,
    out_type=jax.ShapeDtypeStruct((2, NP, 128), jnp.float32),
    scratch_types=[
        pltpu.VMEM((1, CH), jnp.int32),
        pltpu.VMEM((1, CH), jnp.int32),
        pltpu.VMEM((1, CH), jnp.int32),
        pltpu.VMEM((1, CH), jnp.int32),
        pltpu.VMEM((CH, 128), jnp.float32),
        pltpu.VMEM((CH, 128), jnp.float32),
        pltpu.VMEM_SHARED((NP, 128), jnp.float32),
        pltpu.SemaphoreType.DMA,
        pltpu.SemaphoreType.DMA,
        pltpu.SemaphoreType.DMA,
        pltpu.SemaphoreType.DMA,
        pltpu.SemaphoreType.DMA,
        pltpu.SemaphoreType.DMA,
    ],
)
def _sc_aggregate(hs_hbm, rowadj_hbm, col_hbm, agg_hbm, row0, row1, col0,
                  col1, gbuf0, gbuf1, shared, rs0, rs1, cs0, cs1, gs0, gs1):
    c = lax.axis_index("c")
    s = lax.axis_index("s")
    rows = (row0, row1)
    cols = (col0, col1)
    gbufs = (gbuf0, gbuf1)
    rsems = (rs0, rs1)
    csems = (cs0, cs1)
    gsems = (gs0, gs1)

    def row_start(j, b):
        pltpu.async_copy(rowadj_hbm.at[c, s, j], rows[b].at[0], rsems[b])

    def row_wait(b):
        pltpu.make_async_copy(rowadj_hbm.at[c, s, 0], rows[b].at[0],
                              rsems[b]).wait()

    def col_start(j, b):
        pltpu.async_copy(col_hbm.at[s, j], cols[b].at[0], csems[b])

    def col_wait(b):
        pltpu.make_async_copy(col_hbm.at[s, 0], cols[b].at[0],
                              csems[b]).wait()

    def gather_start(b):
        pltpu.async_copy(hs_hbm.at[rows[b].at[0]], gbufs[b], gsems[b])

    def gather_wait(b):
        pltpu.make_async_copy(hs_hbm.at[rows[b].at[0]], gbufs[b],
                              gsems[b]).wait()

    def scatter(b):
        pltpu.sync_copy(gbufs[b], shared.at[cols[b].at[0]], add=True)

    base = s * ROWS_PER_TILE
    row_start(0, 0)
    col_start(0, 0)
    row_start(1, 1)
    col_start(1, 1)
    for p in range(ROWS_PER_TILE // CH):
        pltpu.sync_copy(hs_hbm.at[pl.ds(c * NP + base + p * CH, CH)], gbuf0)
        pltpu.sync_copy(gbuf0, shared.at[pl.ds(base + p * CH, CH)])
    plsc.subcore_barrier()
    row_wait(0)
    gather_start(0)

    def body(g, carry):
        j0 = 2 * g
        row_wait(1)
        gather_start(1)
        gather_wait(0)
        row_start(jnp.minimum(j0 + 2, AGG_CHUNKS - 1), 0)
        col_wait(0)
        scatter(0)
        col_start(jnp.minimum(j0 + 2, AGG_CHUNKS - 1), 0)
        row_wait(0)
        gather_start(0)
        gather_wait(1)
        row_start(jnp.minimum(j0 + 3, AGG_CHUNKS - 1), 1)
        col_wait(1)
        scatter(1)
        col_start(jnp.minimum(j0 + 3, AGG_CHUNKS - 1), 1)
        return carry

    lax.fori_loop(0, AGG_CHUNKS // 2, body, 0)
    gather_wait(0)
    row_wait(1)
    col_wait(0)
    col_wait(1)
    plsc.subcore_barrier()
    for p in range(ROWS_PER_TILE // CH):
        pltpu.sync_copy(shared.at[pl.ds(base + p * CH, CH)], gbuf0)
        pltpu.sync_copy(gbuf0, agg_hbm.at[c, pl.ds(base + p * CH, CH)])


_BLK = 640
_GRID = NP // _BLK


def _tc_pre_body(x_ref, w_ref, dinv_ref, hs_ref):
    hl = jnp.dot(x_ref[...], w_ref[...], preferred_element_type=jnp.float32)
    hs = hl * dinv_ref[...]
    hs_ref[0] = hs[:, :128]
    hs_ref[1] = hs[:, 128:]


def _tc_pre(x, w, dinv_col):
    return pl.pallas_call(
        _tc_pre_body,
        grid=(_GRID,),
        in_specs=[
            pl.BlockSpec((_BLK, D), lambda i: (i, 0)),
            pl.BlockSpec((D, H), lambda i: (0, 0)),
            pl.BlockSpec((_BLK, 1), lambda i: (i, 0)),
        ],
        out_specs=pl.BlockSpec((2, _BLK, 128), lambda i: (0, i, 0)),
        out_shape=jax.ShapeDtypeStruct((2, NP, 128), jnp.float32),
    )(x, w, dinv_col)


def _tc_mid_body(agg_ref, dinv_ref, hprev_ref, a_ref, c_ref, w_ref,
                 h_ref, hs_ref):
    aggf = jnp.concatenate([agg_ref[0], agg_ref[1]], axis=1)
    t = aggf * dinv_ref[...]
    y = t * a_ref[...] + c_ref[...]
    h = jnp.maximum(y, 0.0) + hprev_ref[...]
    h_ref[...] = h
    hl = jnp.dot(h, w_ref[...], preferred_element_type=jnp.float32)
    hs = hl * dinv_ref[...]
    hs_ref[0] = hs[:, :128]
    hs_ref[1] = hs[:, 128:]


def _tc_mid(agg, dinv_col, hprev, a, c, w):
    return pl.pallas_call(
        _tc_mid_body,
        grid=(_GRID,),
        in_specs=[
            pl.BlockSpec((2, _BLK, 128), lambda i: (0, i, 0)),
            pl.BlockSpec((_BLK, 1), lambda i: (i, 0)),
            pl.BlockSpec((_BLK, H), lambda i: (i, 0)),
            pl.BlockSpec((1, H), lambda i: (0, 0)),
            pl.BlockSpec((1, H), lambda i: (0, 0)),
            pl.BlockSpec((H, H), lambda i: (0, 0)),
        ],
        out_specs=[
            pl.BlockSpec((_BLK, H), lambda i: (i, 0)),
            pl.BlockSpec((2, _BLK, 128), lambda i: (0, i, 0)),
        ],
        out_shape=[
            jax.ShapeDtypeStruct((NP, H), jnp.float32),
            jax.ShapeDtypeStruct((2, NP, 128), jnp.float32),
        ],
    )(agg, dinv_col, hprev, a, c, w)


def _tc_part_body(h1_ref, h2_ref, wf12_ref, bf_ref, part_ref):
    wf = wf12_ref[...]
    out = jnp.dot(h1_ref[...], wf[:H], preferred_element_type=jnp.float32)
    out += jnp.dot(h2_ref[...], wf[H:], preferred_element_type=jnp.float32)
    part_ref[...] = out + bf_ref[...]


def _tc_part(h1, h2, wf12, bf):
    return pl.pallas_call(
        _tc_part_body,
        grid=(_GRID,),
        in_specs=[
            pl.BlockSpec((_BLK, H), lambda i: (i, 0)),
            pl.BlockSpec((_BLK, H), lambda i: (i, 0)),
            pl.BlockSpec((2 * H, H), lambda i: (0, 0)),
            pl.BlockSpec((1, H), lambda i: (0, 0)),
        ],
        out_specs=pl.BlockSpec((_BLK, H), lambda i: (i, 0)),
        out_shape=jax.ShapeDtypeStruct((NP, H), jnp.float32),
    )(h1, h2, wf12, bf)


def _tc_final_body(agg_ref, dinv_ref, h2_ref, a_ref, c_ref, part_ref,
                   wf3_ref, out_ref):
    aggf = jnp.concatenate([agg_ref[0], agg_ref[1]], axis=1)
    t = aggf * dinv_ref[...]
    y = t * a_ref[...] + c_ref[...]
    h3 = jnp.maximum(y, 0.0) + h2_ref[...]
    out_ref[...] = part_ref[...] + jnp.dot(
        h3, wf3_ref[...], preferred_element_type=jnp.float32)


def _tc_final(agg, dinv_col, h2, a, c, part, wf3):
    return pl.pallas_call(
        _tc_final_body,
        grid=(_GRID,),
        in_specs=[
            pl.BlockSpec((2, _BLK, 128), lambda i: (0, i, 0)),
            pl.BlockSpec((_BLK, 1), lambda i: (i, 0)),
            pl.BlockSpec((_BLK, H), lambda i: (i, 0)),
            pl.BlockSpec((1, H), lambda i: (0, 0)),
            pl.BlockSpec((1, H), lambda i: (0, 0)),
            pl.BlockSpec((_BLK, H), lambda i: (i, 0)),
            pl.BlockSpec((H, H), lambda i: (0, 0)),
        ],
        out_specs=pl.BlockSpec((_BLK, H), lambda i: (i, 0)),
        out_shape=jax.ShapeDtypeStruct((NP, H), jnp.float32),
    )(agg, dinv_col, h2, a, c, part, wf3)


def kernel(x, edge_index, W0, W1, W2, b0, b1, b2, g0, g1, g2, be0, be1, be2,
           rm0, rm1, rm2, rv0, rv1, rv2, Wf, bf):
    row = edge_index[0].astype(jnp.int32)
    col = edge_index[1].astype(jnp.int32)

    padn = EPAD - E
    padidx = (N + (jnp.arange(padn, dtype=jnp.int32) % (NP - N)))
    row_p = jnp.concatenate([row, padidx])
    col_p = jnp.concatenate([col, padidx])
    col3d = col_p.reshape(NS, AGG_CHUNKS, CH)
    row3d = row_p.reshape(NS, AGG_CHUNKS, CH)
    rowadj = jnp.stack([row3d, row3d + NP])

    x_pad = jnp.concatenate(
        [x, jnp.zeros((NP - N, D), jnp.float32)], axis=0)

    deg_part = _sc_degree(col3d)
    dinv_col = lax.rsqrt(deg_part[0] + deg_part[1] + 1.0)[:, None]

    as_, cs_ = [], []
    for g, rv, rm, be, b in ((g0, rv0, rm0, be0, b0),
                             (g1, rv1, rm1, be1, b1),
                             (g2, rv2, rm2, be2, b2)):
        p = g * lax.rsqrt(rv + EPS)
        as_.append(p[None, :])
        cs_.append((b * p + be - rm * p)[None, :])

    hs = _tc_pre(x_pad, W0, dinv_col)
    hs_flat = hs.reshape(2 * NP, 128)
    agg0 = _sc_aggregate(hs_flat, rowadj, col3d)
    h1, hs1 = _tc_mid(agg0, dinv_col, x_pad, as_[0], cs_[0], W1)
    agg1 = _sc_aggregate(hs1.reshape(2 * NP, 128), rowadj, col3d)
    h2, hs2 = _tc_mid(agg1, dinv_col, h1, as_[1], cs_[1], W2)
    agg2 = _sc_aggregate(hs2.reshape(2 * NP, 128), rowadj, col3d)
    part = _tc_part(h1, h2, Wf[:2 * H], bf[None, :])
    out = _tc_final(agg2, dinv_col, h2, as_[2], cs_[2], part, Wf[2 * H:])
    return out[:N]


# speedup vs baseline: 1.0511x; 1.0511x over previous
import functools

import jax
import jax.numpy as jnp
from jax import lax
from jax.experimental import pallas as pl
from jax.experimental.pallas import tpu as pltpu
from jax.experimental.pallas import tpu_sc as plsc

N = 10000
NP = 10240
E = 160000
EPAD = 163840
D = 256
H = 256
L = 3
EPS = 1e-5

NS = 16
ROWS_PER_TILE = NP // NS
CH = 80
AGG_CHUNKS = EPAD // (NS * CH)
DEG_CHUNKS = AGG_CHUNKS // 2

_MESH = plsc.VectorSubcoreMesh(core_axis_name="c", subcore_axis_name="s")


@functools.partial(
    pl.kernel,
    mesh=_MESH,
    out_type=jax.ShapeDtypeStruct((2, NP), jnp.float32),
    scratch_types=[
        pltpu.VMEM((DEG_CHUNKS, CH), jnp.int32),
        pltpu.VMEM((CH,), jnp.float32),
        pltpu.VMEM((ROWS_PER_TILE,), jnp.float32),
        pltpu.VMEM_SHARED((NP,), jnp.float32),
    ],
)
def _sc_degree(col_hbm, deg_hbm, colv, onesv, wbuf, shared):
    c = lax.axis_index("c")
    s = lax.axis_index("s")
    one = jnp.ones((16,), jnp.float32)
    zero = jnp.zeros((16,), jnp.float32)
    for j in range(CH // 16):
        onesv[pl.ds(16 * j, 16)] = one
    for j in range(ROWS_PER_TILE // 16):
        wbuf[pl.ds(16 * j, 16)] = zero
    base = s * ROWS_PER_TILE
    pltpu.sync_copy(wbuf, shared.at[pl.ds(base, ROWS_PER_TILE)])
    pltpu.sync_copy(col_hbm.at[s, pl.ds(c * DEG_CHUNKS, DEG_CHUNKS)], colv)
    plsc.subcore_barrier()

    def body(j, carry):
        pltpu.sync_copy(onesv, shared.at[colv.at[j]], add=True)
        return carry

    lax.fori_loop(0, DEG_CHUNKS, body, 0)
    plsc.subcore_barrier()
    pltpu.sync_copy(shared.at[pl.ds(base, ROWS_PER_TILE)], wbuf)
    pltpu.sync_copy(wbuf, deg_hbm.at[c, pl.ds(base, ROWS_PER_TILE)])


@functools.partial(
    pl.kernel,
    mesh=_MESH,
    out_type=jax.ShapeDtypeStruct((2, NP, 128), jnp.float32),
    scratch_types=[
        pltpu.VMEM((1, CH), jnp.int32),
        pltpu.VMEM((1, CH), jnp.int32),
        pltpu.VMEM((1, CH), jnp.int32),
        pltpu.VMEM((1, CH), jnp.int32),
        pltpu.VMEM((CH, 128), jnp.float32),
        pltpu.VMEM((CH, 128), jnp.float32),
        pltpu.VMEM_SHARED((NP, 128), jnp.float32),
        pltpu.SemaphoreType.DMA,
        pltpu.SemaphoreType.DMA,
        pltpu.SemaphoreType.DMA,
        pltpu.SemaphoreType.DMA,
        pltpu.SemaphoreType.DMA,
        pltpu.SemaphoreType.DMA,
    ],
)
def _sc_aggregate(hs_hbm, rowadj_hbm, col_hbm, agg_hbm, row0, row1, col0,
                  col1, gbuf0, gbuf1, shared, rs0, rs1, cs0, cs1, gs0, gs1):
    c = lax.axis_index("c")
    s = lax.axis_index("s")
    rows = (row0, row1)
    cols = (col0, col1)
    gbufs = (gbuf0, gbuf1)
    rsems = (rs0, rs1)
    csems = (cs0, cs1)
    gsems = (gs0, gs1)

    def row_start(j, b):
        pltpu.async_copy(rowadj_hbm.at[c, s, j], rows[b].at[0], rsems[b])

    def row_wait(b):
        pltpu.make_async_copy(rowadj_hbm.at[c, s, 0], rows[b].at[0],
                              rsems[b]).wait()

    def col_start(j, b):
        pltpu.async_copy(col_hbm.at[s, j], cols[b].at[0], csems[b])

    def col_wait(b):
        pltpu.make_async_copy(col_hbm.at[s, 0], cols[b].at[0],
                              csems[b]).wait()

    def gather_start(b):
        pltpu.async_copy(hs_hbm.at[rows[b].at[0]], gbufs[b], gsems[b])

    def gather_wait(b):
        pltpu.make_async_copy(hs_hbm.at[rows[b].at[0]], gbufs[b],
                              gsems[b]).wait()

    def scatter(b):
        pltpu.sync_copy(gbufs[b], shared.at[cols[b].at[0]], add=True)

    base = s * ROWS_PER_TILE
    row_start(0, 0)
    col_start(0, 0)
    row_start(1, 1)
    col_start(1, 1)
    pltpu.sync_copy(hs_hbm.at[pl.ds(c * NP + base, ROWS_PER_TILE)],
                    shared.at[pl.ds(base, ROWS_PER_TILE)])
    plsc.subcore_barrier()
    row_wait(0)
    gather_start(0)

    def body(g, carry):
        j0 = 2 * g
        row_wait(1)
        gather_start(1)
        gather_wait(0)
        row_start(jnp.minimum(j0 + 2, AGG_CHUNKS - 1), 0)
        col_wait(0)
        scatter(0)
        col_start(jnp.minimum(j0 + 2, AGG_CHUNKS - 1), 0)
        row_wait(0)
        gather_start(0)
        gather_wait(1)
        row_start(jnp.minimum(j0 + 3, AGG_CHUNKS - 1), 1)
        col_wait(1)
        scatter(1)
        col_start(jnp.minimum(j0 + 3, AGG_CHUNKS - 1), 1)
        return carry

    lax.fori_loop(0, AGG_CHUNKS // 2, body, 0)
    gather_wait(0)
    row_wait(1)
    col_wait(0)
    col_wait(1)
    plsc.subcore_barrier()
    pltpu.sync_copy(shared.at[pl.ds(base, ROWS_PER_TILE)],
                    agg_hbm.at[c, pl.ds(base, ROWS_PER_TILE)])


_BLK = 640
_GRID = NP // _BLK


def _tc_pre_body(x_ref, w_ref, dinv_ref, hs_ref):
    hl = jnp.dot(x_ref[...], w_ref[...], preferred_element_type=jnp.float32)
    hs = hl * dinv_ref[...]
    hs_ref[0] = hs[:, :128]
    hs_ref[1] = hs[:, 128:]


def _tc_pre(x, w, dinv_col):
    return pl.pallas_call(
        _tc_pre_body,
        grid=(_GRID,),
        in_specs=[
            pl.BlockSpec((_BLK, D), lambda i: (i, 0)),
            pl.BlockSpec((D, H), lambda i: (0, 0)),
            pl.BlockSpec((_BLK, 1), lambda i: (i, 0)),
        ],
        out_specs=pl.BlockSpec((2, _BLK, 128), lambda i: (0, i, 0)),
        out_shape=jax.ShapeDtypeStruct((2, NP, 128), jnp.float32),
    )(x, w, dinv_col)


def _tc_mid_body(agg_ref, dinv_ref, hprev_ref, a_ref, c_ref, w_ref,
                 h_ref, hs_ref):
    aggf = jnp.concatenate([agg_ref[0], agg_ref[1]], axis=1)
    t = aggf * dinv_ref[...]
    y = t * a_ref[...] + c_ref[...]
    h = jnp.maximum(y, 0.0) + hprev_ref[...]
    h_ref[...] = h
    hl = jnp.dot(h, w_ref[...], preferred_element_type=jnp.float32)
    hs = hl * dinv_ref[...]
    hs_ref[0] = hs[:, :128]
    hs_ref[1] = hs[:, 128:]


def _tc_mid(agg, dinv_col, hprev, a, c, w):
    return pl.pallas_call(
        _tc_mid_body,
        grid=(_GRID,),
        in_specs=[
            pl.BlockSpec((2, _BLK, 128), lambda i: (0, i, 0)),
            pl.BlockSpec((_BLK, 1), lambda i: (i, 0)),
            pl.BlockSpec((_BLK, H), lambda i: (i, 0)),
            pl.BlockSpec((1, H), lambda i: (0, 0)),
            pl.BlockSpec((1, H), lambda i: (0, 0)),
            pl.BlockSpec((H, H), lambda i: (0, 0)),
        ],
        out_specs=[
            pl.BlockSpec((_BLK, H), lambda i: (i, 0)),
            pl.BlockSpec((2, _BLK, 128), lambda i: (0, i, 0)),
        ],
        out_shape=[
            jax.ShapeDtypeStruct((NP, H), jnp.float32),
            jax.ShapeDtypeStruct((2, NP, 128), jnp.float32),
        ],
    )(agg, dinv_col, hprev, a, c, w)


def _tc_part_body(h1_ref, h2_ref, wf12_ref, bf_ref, part_ref):
    wf = wf12_ref[...]
    out = jnp.dot(h1_ref[...], wf[:H], preferred_element_type=jnp.float32)
    out += jnp.dot(h2_ref[...], wf[H:], preferred_element_type=jnp.float32)
    part_ref[...] = out + bf_ref[...]


def _tc_part(h1, h2, wf12, bf):
    return pl.pallas_call(
        _tc_part_body,
        grid=(_GRID,),
        in_specs=[
            pl.BlockSpec((_BLK, H), lambda i: (i, 0)),
            pl.BlockSpec((_BLK, H), lambda i: (i, 0)),
            pl.BlockSpec((2 * H, H), lambda i: (0, 0)),
            pl.BlockSpec((1, H), lambda i: (0, 0)),
        ],
        out_specs=pl.BlockSpec((_BLK, H), lambda i: (i, 0)),
        out_shape=jax.ShapeDtypeStruct((NP, H), jnp.float32),
    )(h1, h2, wf12, bf)


def _tc_final_body(agg_ref, dinv_ref, h2_ref, a_ref, c_ref, part_ref,
                   wf3_ref, out_ref):
    aggf = jnp.concatenate([agg_ref[0], agg_ref[1]], axis=1)
    t = aggf * dinv_ref[...]
    y = t * a_ref[...] + c_ref[...]
    h3 = jnp.maximum(y, 0.0) + h2_ref[...]
    out_ref[...] = part_ref[...] + jnp.dot(
        h3, wf3_ref[...], preferred_element_type=jnp.float32)


def _tc_final(agg, dinv_col, h2, a, c, part, wf3):
    return pl.pallas_call(
        _tc_final_body,
        grid=(_GRID,),
        in_specs=[
            pl.BlockSpec((2, _BLK, 128), lambda i: (0, i, 0)),
            pl.BlockSpec((_BLK, 1), lambda i: (i, 0)),
            pl.BlockSpec((_BLK, H), lambda i: (i, 0)),
            pl.BlockSpec((1, H), lambda i: (0, 0)),
            pl.BlockSpec((1, H), lambda i: (0, 0)),
            pl.BlockSpec((_BLK, H), lambda i: (i, 0)),
            pl.BlockSpec((H, H), lambda i: (0, 0)),
        ],
        out_specs=pl.BlockSpec((_BLK, H), lambda i: (i, 0)),
        out_shape=jax.ShapeDtypeStruct((N, H), jnp.float32),
    )(agg, dinv_col, h2, a, c, part, wf3)


def kernel(x, edge_index, W0, W1, W2, b0, b1, b2, g0, g1, g2, be0, be1, be2,
           rm0, rm1, rm2, rv0, rv1, rv2, Wf, bf):
    row = edge_index[0].astype(jnp.int32)
    col = edge_index[1].astype(jnp.int32)

    padn = EPAD - E
    padidx = (N + (jnp.arange(padn, dtype=jnp.int32) % (NP - N)))
    row_p = jnp.concatenate([row, padidx])
    col_p = jnp.concatenate([col, padidx])
    col3d = col_p.reshape(NS, AGG_CHUNKS, CH)
    row3d = row_p.reshape(NS, AGG_CHUNKS, CH)
    rowadj = jnp.stack([row3d, row3d + NP])

    x_pad = jnp.concatenate(
        [x, jnp.zeros((NP - N, D), jnp.float32)], axis=0)

    deg_part = _sc_degree(col3d)
    dinv_col = lax.rsqrt(deg_part[0] + deg_part[1] + 1.0)[:, None]

    as_, cs_ = [], []
    for g, rv, rm, be, b in ((g0, rv0, rm0, be0, b0),
                             (g1, rv1, rm1, be1, b1),
                             (g2, rv2, rm2, be2, b2)):
        p = g * lax.rsqrt(rv + EPS)
        as_.append(p[None, :])
        cs_.append((b * p + be - rm * p)[None, :])

    hs = _tc_pre(x_pad, W0, dinv_col)
    hs_flat = hs.reshape(2 * NP, 128)
    agg0 = _sc_aggregate(hs_flat, rowadj, col3d)
    h1, hs1 = _tc_mid(agg0, dinv_col, x_pad, as_[0], cs_[0], W1)
    agg1 = _sc_aggregate(hs1.reshape(2 * NP, 128), rowadj, col3d)
    h2, hs2 = _tc_mid(agg1, dinv_col, h1, as_[1], cs_[1], W2)
    agg2 = _sc_aggregate(hs2.reshape(2 * NP, 128), rowadj, col3d)
    part = _tc_part(h1, h2, Wf[:2 * H], bf[None, :])
    return _tc_final(agg2, dinv_col, h2, as_[2], cs_[2], part, Wf[2 * H:])


# speedup vs baseline: 1.0513x; 1.0002x over previous
import functools

import jax
import jax.numpy as jnp
from jax import lax
from jax.experimental import pallas as pl
from jax.experimental.pallas import tpu as pltpu
from jax.experimental.pallas import tpu_sc as plsc

N = 10000
NP = 10240
E = 160000
EPAD = 163840
D = 256
H = 256
L = 3
EPS = 1e-5

NS = 16
ROWS_PER_TILE = NP // NS
CH = 80
AGG_CHUNKS = EPAD // (NS * CH)
DEG_CHUNKS = AGG_CHUNKS // 2

_MESH = plsc.VectorSubcoreMesh(core_axis_name="c", subcore_axis_name="s")


@functools.partial(
    pl.kernel,
    mesh=_MESH,
    out_type=jax.ShapeDtypeStruct((2, NP), jnp.float32),
    scratch_types=[
        pltpu.VMEM((DEG_CHUNKS, CH), jnp.int32),
        pltpu.VMEM((CH,), jnp.float32),
        pltpu.VMEM((ROWS_PER_TILE,), jnp.float32),
        pltpu.VMEM_SHARED((NP,), jnp.float32),
    ],
)
def _sc_degree(col_hbm, deg_hbm, colv, onesv, wbuf, shared):
    c = lax.axis_index("c")
    s = lax.axis_index("s")
    one = jnp.ones((16,), jnp.float32)
    zero = jnp.zeros((16,), jnp.float32)
    for j in range(CH // 16):
        onesv[pl.ds(16 * j, 16)] = one
    for j in range(ROWS_PER_TILE // 16):
        wbuf[pl.ds(16 * j, 16)] = zero
    base = s * ROWS_PER_TILE
    pltpu.sync_copy(wbuf, shared.at[pl.ds(base, ROWS_PER_TILE)])
    pltpu.sync_copy(col_hbm.at[s, pl.ds(c * DEG_CHUNKS, DEG_CHUNKS)], colv)
    plsc.subcore_barrier()

    def body(j, carry):
        pltpu.sync_copy(onesv, shared.at[colv.at[j]], add=True)
        return carry

    lax.fori_loop(0, DEG_CHUNKS, body, 0)
    plsc.subcore_barrier()
    pltpu.sync_copy(shared.at[pl.ds(base, ROWS_PER_TILE)], wbuf)
    pltpu.sync_copy(wbuf, deg_hbm.at[c, pl.ds(base, ROWS_PER_TILE)])


@functools.partial(
    pl.kernel,
    mesh=_MESH,
    out_type=jax.ShapeDtypeStruct((2, NP, 128), jnp.float32),
    scratch_types=[
        pltpu.VMEM((1, CH), jnp.int32),
        pltpu.VMEM((1, CH), jnp.int32),
        pltpu.VMEM((1, CH), jnp.int32),
        pltpu.VMEM((1, CH), jnp.int32),
        pltpu.VMEM((CH, 128), jnp.float32),
        pltpu.VMEM((CH, 128), jnp.float32),
        pltpu.VMEM_SHARED((NP, 128), jnp.float32),
        pltpu.SemaphoreType.DMA,
        pltpu.SemaphoreType.DMA,
        pltpu.SemaphoreType.DMA,
        pltpu.SemaphoreType.DMA,
        pltpu.SemaphoreType.DMA,
        pltpu.SemaphoreType.DMA,
    ],
)
def _sc_aggregate(hs_hbm, rowadj_hbm, col_hbm, agg_hbm, row0, row1, col0,
                  col1, gbuf0, gbuf1, shared, rs0, rs1, cs0, cs1, gs0, gs1):
    c = lax.axis_index("c")
    s = lax.axis_index("s")
    rows = (row0, row1)
    cols = (col0, col1)
    gbufs = (gbuf0, gbuf1)
    rsems = (rs0, rs1)
    csems = (cs0, cs1)
    gsems = (gs0, gs1)

    def row_start(j, b):
        pltpu.async_copy(rowadj_hbm.at[c, s, j], rows[b].at[0], rsems[b])

    def row_wait(b):
        pltpu.make_async_copy(rowadj_hbm.at[c, s, 0], rows[b].at[0],
                              rsems[b]).wait()

    def col_start(j, b):
        pltpu.async_copy(col_hbm.at[s, j], cols[b].at[0], csems[b])

    def col_wait(b):
        pltpu.make_async_copy(col_hbm.at[s, 0], cols[b].at[0],
                              csems[b]).wait()

    def gather_start(b):
        pltpu.async_copy(hs_hbm.at[rows[b].at[0]], gbufs[b], gsems[b])

    def gather_wait(b):
        pltpu.make_async_copy(hs_hbm.at[rows[b].at[0]], gbufs[b],
                              gsems[b]).wait()

    def scatter(b):
        pltpu.sync_copy(gbufs[b], shared.at[cols[b].at[0]], add=True)

    base = s * ROWS_PER_TILE
    row_start(0, 0)
    col_start(0, 0)
    row_start(1, 1)
    col_start(1, 1)
    pltpu.sync_copy(hs_hbm.at[pl.ds(c * NP + base, ROWS_PER_TILE)],
                    shared.at[pl.ds(base, ROWS_PER_TILE)])
    plsc.subcore_barrier()
    row_wait(0)
    gather_start(0)

    def body(g, carry):
        j0 = 2 * g
        row_wait(1)
        gather_start(1)
        gather_wait(0)
        row_start(jnp.minimum(j0 + 2, AGG_CHUNKS - 1), 0)
        col_wait(0)
        scatter(0)
        col_start(jnp.minimum(j0 + 2, AGG_CHUNKS - 1), 0)
        row_wait(0)
        gather_start(0)
        gather_wait(1)
        row_start(jnp.minimum(j0 + 3, AGG_CHUNKS - 1), 1)
        col_wait(1)
        scatter(1)
        col_start(jnp.minimum(j0 + 3, AGG_CHUNKS - 1), 1)
        return carry

    lax.fori_loop(0, AGG_CHUNKS // 2, body, 0)
    gather_wait(0)
    row_wait(1)
    col_wait(0)
    col_wait(1)
    plsc.subcore_barrier()
    pltpu.sync_copy(shared.at[pl.ds(base, ROWS_PER_TILE)],
                    agg_hbm.at[c, pl.ds(base, ROWS_PER_TILE)])


_BLK = 640
_GRID = NP // _BLK


def _tc_pre_body(x_ref, w_ref, dinv_ref, hs_ref):
    hl = jnp.dot(x_ref[...], w_ref[...], preferred_element_type=jnp.float32)
    hs = hl * dinv_ref[...]
    hs_ref[0] = hs[:, :128]
    hs_ref[1] = hs[:, 128:]


def _tc_pre(x, w, dinv_col):
    return pl.pallas_call(
        _tc_pre_body,
        grid=(_GRID,),
        in_specs=[
            pl.BlockSpec((_BLK, D), lambda i: (i, 0)),
            pl.BlockSpec((D, H), lambda i: (0, 0)),
            pl.BlockSpec((_BLK, 1), lambda i: (i, 0)),
        ],
        out_specs=pl.BlockSpec((2, _BLK, 128), lambda i: (0, i, 0)),
        out_shape=jax.ShapeDtypeStruct((2, NP, 128), jnp.float32),
    )(x, w, dinv_col)


def _tc_mid_body(agg_ref, dinv_ref, hprev_ref, a_ref, c_ref, w_ref,
                 h_ref, hs_ref):
    aggf = jnp.concatenate([agg_ref[0], agg_ref[1]], axis=1)
    t = aggf * dinv_ref[...]
    y = t * a_ref[...] + c_ref[...]
    h = jnp.maximum(y, 0.0) + hprev_ref[...]
    h_ref[...] = h
    hl = jnp.dot(h, w_ref[...], preferred_element_type=jnp.float32)
    hs = hl * dinv_ref[...]
    hs_ref[0] = hs[:, :128]
    hs_ref[1] = hs[:, 128:]


def _tc_mid(agg, dinv_col, hprev, a, c, w):
    return pl.pallas_call(
        _tc_mid_body,
        grid=(_GRID,),
        in_specs=[
            pl.BlockSpec((2, _BLK, 128), lambda i: (0, i, 0)),
            pl.BlockSpec((_BLK, 1), lambda i: (i, 0)),
            pl.BlockSpec((_BLK, H), lambda i: (i, 0)),
            pl.BlockSpec((1, H), lambda i: (0, 0)),
            pl.BlockSpec((1, H), lambda i: (0, 0)),
            pl.BlockSpec((H, H), lambda i: (0, 0)),
        ],
        out_specs=[
            pl.BlockSpec((_BLK, H), lambda i: (i, 0)),
            pl.BlockSpec((2, _BLK, 128), lambda i: (0, i, 0)),
        ],
        out_shape=[
            jax.ShapeDtypeStruct((NP, H), jnp.float32),
            jax.ShapeDtypeStruct((2, NP, 128), jnp.float32),
        ],
    )(agg, dinv_col, hprev, a, c, w)


def _tc_final_body(agg_ref, dinv_ref, h2_ref, a_ref, c_ref, h1_ref, wf_ref,
                   bf_ref, out_ref):
    aggf = jnp.concatenate([agg_ref[0], agg_ref[1]], axis=1)
    t = aggf * dinv_ref[...]
    y = t * a_ref[...] + c_ref[...]
    h2 = h2_ref[...]
    h3 = jnp.maximum(y, 0.0) + h2
    wf = wf_ref[...]
    out = jnp.dot(h1_ref[...], wf[:H], preferred_element_type=jnp.float32)
    out += jnp.dot(h2, wf[H:2 * H], preferred_element_type=jnp.float32)
    out += jnp.dot(h3, wf[2 * H:], preferred_element_type=jnp.float32)
    out_ref[...] = out + bf_ref[...]


def _tc_final(agg, dinv_col, h2, a, c, h1, wf, bf):
    return pl.pallas_call(
        _tc_final_body,
        grid=(_GRID,),
        in_specs=[
            pl.BlockSpec((2, _BLK, 128), lambda i: (0, i, 0)),
            pl.BlockSpec((_BLK, 1), lambda i: (i, 0)),
            pl.BlockSpec((_BLK, H), lambda i: (i, 0)),
            pl.BlockSpec((1, H), lambda i: (0, 0)),
            pl.BlockSpec((1, H), lambda i: (0, 0)),
            pl.BlockSpec((_BLK, H), lambda i: (i, 0)),
            pl.BlockSpec((L * H, H), lambda i: (0, 0)),
            pl.BlockSpec((1, H), lambda i: (0, 0)),
        ],
        out_specs=pl.BlockSpec((_BLK, H), lambda i: (i, 0)),
        out_shape=jax.ShapeDtypeStruct((N, H), jnp.float32),
    )(agg, dinv_col, h2, a, c, h1, wf, bf)


def kernel(x, edge_index, W0, W1, W2, b0, b1, b2, g0, g1, g2, be0, be1, be2,
           rm0, rm1, rm2, rv0, rv1, rv2, Wf, bf):
    row = edge_index[0].astype(jnp.int32)
    col = edge_index[1].astype(jnp.int32)

    padn = EPAD - E
    padidx = (N + (jnp.arange(padn, dtype=jnp.int32) % (NP - N)))
    row_p = jnp.concatenate([row, padidx])
    col_p = jnp.concatenate([col, padidx])
    col3d = col_p.reshape(NS, AGG_CHUNKS, CH)
    row3d = row_p.reshape(NS, AGG_CHUNKS, CH)
    rowadj = jnp.stack([row3d, row3d + NP])

    x_pad = jnp.concatenate(
        [x, jnp.zeros((NP - N, D), jnp.float32)], axis=0)

    deg_part = _sc_degree(col3d)
    dinv_col = lax.rsqrt(deg_part[0] + deg_part[1] + 1.0)[:, None]

    as_, cs_ = [], []
    for g, rv, rm, be, b in ((g0, rv0, rm0, be0, b0),
                             (g1, rv1, rm1, be1, b1),
                             (g2, rv2, rm2, be2, b2)):
        p = g * lax.rsqrt(rv + EPS)
        as_.append(p[None, :])
        cs_.append((b * p + be - rm * p)[None, :])

    hs = _tc_pre(x_pad, W0, dinv_col)
    hs_flat = hs.reshape(2 * NP, 128)
    agg0 = _sc_aggregate(hs_flat, rowadj, col3d)
    h1, hs1 = _tc_mid(agg0, dinv_col, x_pad, as_[0], cs_[0], W1)
    agg1 = _sc_aggregate(hs1.reshape(2 * NP, 128), rowadj, col3d)
    h2, hs2 = _tc_mid(agg1, dinv_col, h1, as_[1], cs_[1], W2)
    agg2 = _sc_aggregate(hs2.reshape(2 * NP, 128), rowadj, col3d)
    return _tc_final(agg2, dinv_col, h2, as_[2], cs_[2], h1, Wf, bf[None, :])
